# Initial kernel scaffold; baseline (speedup 1.0000x reference)
import jax
import jax.numpy as jnp
from jax.experimental import pallas as pl


def kernel(X, he, W1, b1, W2, b2):
    raise NotImplementedError("write your pallas kernel here")


# speedup vs baseline: 2.8328x; 2.8328x over previous
import functools

import jax
import jax.numpy as jnp
from jax import lax
from jax.experimental import pallas as pl
from jax.experimental.pallas import tpu as pltpu

N = 100000
K = 8

_ROW_BLK = 1000


def _mm1_body(x_ref, w_ref, b_ref, s_ref, o_ref):
    h = jnp.dot(x_ref[...], w_ref[...],
                preferred_element_type=jnp.float32,
                precision=jax.lax.Precision.HIGHEST) + b_ref[...]
    o_ref[...] = h * s_ref[...]


def _mm2_body(a_ref, w_ref, b_ref, s_ref, o_ref):
    t = jnp.maximum(a_ref[...] * s_ref[...], 0.0)
    h = jnp.dot(t, w_ref[...],
                preferred_element_type=jnp.float32,
                precision=jax.lax.Precision.HIGHEST) + b_ref[...]
    o_ref[...] = h * s_ref[...]


def _scale_body(a_ref, s_ref, o_ref):
    o_ref[...] = a_ref[...] * s_ref[...]


def _tc_mm1(X, W, b, inv):
    C_in, C_out = W.shape
    return pl.pallas_call(
        _mm1_body,
        grid=(N // _ROW_BLK,),
        in_specs=[
            pl.BlockSpec((_ROW_BLK, C_in), lambda i: (i, 0)),
            pl.BlockSpec((C_in, C_out), lambda i: (0, 0)),
            pl.BlockSpec((1, C_out), lambda i: (0, 0)),
            pl.BlockSpec((_ROW_BLK, 1), lambda i: (i, 0)),
        ],
        out_specs=pl.BlockSpec((_ROW_BLK, C_out), lambda i: (i, 0)),
        out_shape=jax.ShapeDtypeStruct((N, C_out), jnp.float32),
    )(X, W, b.reshape(1, C_out), inv.reshape(N, 1))


def _tc_mm2(A, W, b, inv):
    C_in, C_out = W.shape
    return pl.pallas_call(
        _mm2_body,
        grid=(N // _ROW_BLK,),
        in_specs=[
            pl.BlockSpec((_ROW_BLK, C_in), lambda i: (i, 0)),
            pl.BlockSpec((C_in, C_out), lambda i: (0, 0)),
            pl.BlockSpec((1, C_out), lambda i: (0, 0)),
            pl.BlockSpec((_ROW_BLK, 1), lambda i: (i, 0)),
        ],
        out_specs=pl.BlockSpec((_ROW_BLK, C_out), lambda i: (i, 0)),
        out_shape=jax.ShapeDtypeStruct((N, C_out), jnp.float32),
    )(A, W, b.reshape(1, C_out), inv.reshape(N, 1))


def _tc_scale(A, inv):
    C = A.shape[1]
    return pl.pallas_call(
        _scale_body,
        grid=(N // _ROW_BLK,),
        in_specs=[
            pl.BlockSpec((_ROW_BLK, C), lambda i: (i, 0)),
            pl.BlockSpec((_ROW_BLK, 1), lambda i: (i, 0)),
        ],
        out_specs=pl.BlockSpec((_ROW_BLK, C), lambda i: (i, 0)),
        out_shape=jax.ShapeDtypeStruct((N, C), jnp.float32),
    )(A, inv.reshape(N, 1))


def _build_edges(X, he):
    feats = X[he]
    sq = jnp.sum(feats * feats, axis=-1)
    dots = jnp.einsum('mkc,mlc->mkl', feats, feats)
    d = sq[:, :, None] + sq[:, None, :] - 2.0 * dots
    idx = jnp.argmax(d.reshape(he.shape[0], -1), axis=1)
    i = idx // K
    j = idx % K
    src = jnp.take_along_axis(he, i[:, None], axis=1)[:, 0]
    dst = jnp.take_along_axis(he, j[:, None], axis=1)[:, 0]
    return src, dst


def _smooth_raw(P, src, dst):
    return P.at[dst].add(P[src]).at[src].add(P[dst])


def kernel(X, he, W1, b1, W2, b2):
    src, dst = _build_edges(X, he)
    cnt = jnp.ones((N,), jnp.float32).at[src].add(1.0).at[dst].add(1.0)
    inv = lax.rsqrt(cnt)

    P1 = _tc_mm1(X, W1, b1, inv)
    A1 = _smooth_raw(P1, src, dst)
    P2 = _tc_mm2(A1, W2, b2, inv)
    A2 = _smooth_raw(P2, src, dst)
    return _tc_scale(A2, inv)
